# baseline (speedup 1.0000x reference)
import functools

import jax
import jax.numpy as jnp
from jax import lax
from jax.experimental import pallas as pl
from jax.experimental.pallas import tpu as pltpu
from jax.experimental.pallas import tpu_sc as plsc

B = 16384
D = 32
K = 20
L = 16
NC = 2
NS = 16
NW = NC * NS
BPW = B // NW
CH = 64
NG = BPW // CH
GI = CH * K // 128


def _sc_partials(x2d, xn2d, x_original, wpad, mpad, E):
    mesh = plsc.VectorSubcoreMesh(
        core_axis_name="c", subcore_axis_name="s",
        num_cores=NC, num_subcores=NS)

    scratch = []
    for _ in range(2):
        scratch += [
            pltpu.VMEM((1, CH), jnp.int32),
            pltpu.VMEM((GI, 128), jnp.int32),
            pltpu.VMEM((CH, D), jnp.float32),
            pltpu.VMEM((CH * K, D), jnp.float32),
            pltpu.VMEM((CH, D), jnp.float32),
            pltpu.VMEM((CH, D), jnp.float32),
            pltpu.VMEM((CH, D), jnp.float32),
        ]
    scratch += [
        pltpu.VMEM((CH, D), jnp.float32),
        pltpu.VMEM((1, L), jnp.float32),
        pltpu.VMEM((1, L), jnp.float32),
        pltpu.SemaphoreType.DMA,
        pltpu.SemaphoreType.DMA,
    ]

    @functools.partial(
        pl.kernel,
        out_type=(jax.ShapeDtypeStruct((NW, L), jnp.float32),
                  jax.ShapeDtypeStruct((NW, L), jnp.float32)),
        mesh=mesh,
        scratch_types=scratch,
        compiler_params=pltpu.CompilerParams(
            use_tc_tiling_on_sc=False, needs_layout_passes=False),
    )
    def body(x_ref, xn_ref, xo_ref, w_ref, m_ref, e_ref, out1_ref, out2_ref,
             xi0, xni0, ex0, nb0, xob0, wpb0, mpb0,
             xi1, xni1, ex1, nb1, xob1, wpb1, mpb1,
             coef, res1, res2, sem0, sem1):
        slots = ((xi0, xni0, ex0, nb0, xob0, wpb0, mpb0, sem0),
                 (xi1, xni1, ex1, nb1, xob1, wpb1, mpb1, sem1))
        wid = lax.axis_index("s") * NC + lax.axis_index("c")

        def issue(g, slot):
            xi, xni, ex, nb, xob, wpb, mpb, sem = slots[slot]
            row0 = wid * BPW + g * CH
            pltpu.sync_copy(x_ref.at[pl.ds(wid * NG + g, 1)], xi)
            pltpu.sync_copy(xn_ref.at[pl.ds(wid * (NG * GI) + g * GI, GI)],
                            xni)
            for j in range(GI):
                for h in range(8):
                    v = xni[j, pl.ds(h * L, L)]
                    xni[j, pl.ds(h * L, L)] = (
                        (v & -8192) + ((v & 2047) << 2) + ((v >> 11) & 3))
            for h in range(CH // L):
                v = xi[0, pl.ds(h * L, L)]
                xi[0, pl.ds(h * L, L)] = (
                    (v & -8192) + ((v & 2047) << 2) + ((v >> 11) & 3))
            pltpu.sync_copy(xo_ref.at[pl.ds(row0, CH)], xob)
            pltpu.sync_copy(w_ref.at[pl.ds(row0, CH)], wpb)
            pltpu.sync_copy(m_ref.at[pl.ds(row0, CH)], mpb)
            descs = [
                pltpu.async_copy(e_ref.at[xni.at[j]],
                                 nb.at[pl.ds(j * 128, 128)], sem)
                for j in range(GI)
            ]
            descs.append(pltpu.async_copy(e_ref.at[xi.at[0]], ex, sem))
            return descs

        def compute(slot, accs):
            _, _, ex, nb, xob, wpb, mpb, _ = slots[slot]

            def cbody(r, carry):
                for h in range(2):
                    wv = wpb[r, pl.ds(h * L, L)]
                    mv = mpb[r, pl.ds(h * L, L)]
                    coef[r, pl.ds(h * L, L)] = wv * mv * mv
                return carry
            lax.fori_loop(0, CH, cbody, 0, unroll=4)

            def bbody(b, accs):
                a1_0, a1_1 = accs[0], accs[1]
                a2 = list(accs[2:])
                e0 = ex[b, pl.ds(0, L)]
                e1 = ex[b, pl.ds(L, L)]
                o0 = xob[b, pl.ds(0, L)]
                o1 = xob[b, pl.ds(L, L)]
                d0 = e0 - o0
                d1 = e1 - o1
                a1_0 = a1_0 + d0 * d0
                a1_1 = a1_1 + d1 * d1
                bb = jnp.broadcast_to(b, (L,))
                for k in range(K):
                    kk = jnp.full((L,), k, jnp.int32)
                    cv = plsc.load_gather(coef, [bb, kk])
                    r = b * K + k
                    n0 = nb[r, pl.ds(0, L)]
                    n1 = nb[r, pl.ds(L, L)]
                    t0 = e0 - n0
                    t1 = e1 - n1
                    a2[k % 4] = a2[k % 4] + t0 * t0 * cv
                    a2[4 + k % 4] = a2[4 + k % 4] + t1 * t1 * cv
                return (a1_0, a1_1) + tuple(a2)

            return lax.fori_loop(0, CH, bbody, accs)

        z = jnp.zeros((L,), jnp.float32)
        accs = (z,) * 10
        descs = issue(0, 0)
        for g in range(NG):
            slot = g & 1
            nxt = issue(g + 1, slot ^ 1) if g + 1 < NG else None
            for dsc in descs:
                dsc.wait()
            accs = compute(slot, accs)
            descs = nxt

        a1 = accs[0] + accs[1]
        a2 = accs[2]
        for v in accs[3:]:
            a2 = a2 + v
        res1[0, :] = a1
        res2[0, :] = a2
        pltpu.sync_copy(res1, out1_ref.at[pl.ds(wid, 1)])
        pltpu.sync_copy(res2, out2_ref.at[pl.ds(wid, 1)])

    return body(x2d, xn2d, x_original, wpad, mpad, E)


def _transpose_body(et_ref, out_ref):
    r = lax.broadcasted_iota(jnp.int32, (32, 32), 0)
    c = lax.broadcasted_iota(jnp.int32, (32, 32), 1)
    ident = (r == c).astype(jnp.float32)
    for u in range(4):
        out_ref[:, pl.ds(u * 32, 32)] = lax.dot_general(
            et_ref[:, pl.ds(u * 2048, 2048)], ident,
            (((0,), (0,)), ((), ())),
            preferred_element_type=jnp.float32)


def _transpose(ET):
    return pl.pallas_call(
        _transpose_body,
        grid=(123,),
        in_specs=[pl.BlockSpec((32, 8192), lambda j: (0, j))],
        out_specs=pl.BlockSpec((2048, 128), lambda j: (j, 0)),
        out_shape=jax.ShapeDtypeStruct((251904, 128), jnp.float32),
        compiler_params=pltpu.CompilerParams(
            dimension_semantics=("parallel",)),
    )(ET)


def _finish_body(alpha_ref, p1_ref, p2_ref, out_ref):
    s1 = jnp.sum(p1_ref[...])
    s2 = jnp.sum(p2_ref[...])
    out_ref[0, 0] = (alpha_ref[0] * s1 + s2) * (1.0 / B)


def _finish(alpha, p1, p2):
    return pl.pallas_call(
        _finish_body,
        out_shape=jax.ShapeDtypeStruct((1, 1), jnp.float32),
        in_specs=[
            pl.BlockSpec(memory_space=pltpu.SMEM),
            pl.BlockSpec(memory_space=pltpu.VMEM),
            pl.BlockSpec(memory_space=pltpu.VMEM),
        ],
        out_specs=pl.BlockSpec(memory_space=pltpu.SMEM),
    )(alpha, p1, p2)


def kernel(x, x_original, x_neighbor, weight, alpha, num_neighbor, mask, E):
    del num_neighbor
    x2d = x.reshape(B // CH, CH).astype(jnp.int32)
    xn2d = x_neighbor.reshape(B * K // 128, 128).astype(jnp.int32)
    wpad = jnp.pad(weight, ((0, 0), (0, D - K)))
    mpad = jnp.pad(mask, ((0, 0), (0, D - K)))
    E2 = _transpose(E.T).reshape(1007616, 32)
    p1, p2 = _sc_partials(x2d, xn2d, x_original, wpad, mpad, E2)
    out = _finish(alpha.reshape(1), p1, p2)
    return out[0, 0]


# speedup vs baseline: 1.0155x; 1.0155x over previous
import functools

import jax
import jax.numpy as jnp
from jax import lax
from jax.experimental import pallas as pl
from jax.experimental.pallas import tpu as pltpu
from jax.experimental.pallas import tpu_sc as plsc

B = 16384
D = 32
K = 20
L = 16
NC = 2
NS = 16
NW = NC * NS
BPW = B // NW
CH = 64
NG = BPW // CH
GI = CH * K // 128


def _sc_partials(x2d, xn2d, x_original, wpad, mpad, E):
    mesh = plsc.VectorSubcoreMesh(
        core_axis_name="c", subcore_axis_name="s",
        num_cores=NC, num_subcores=NS)

    scratch = []
    for _ in range(2):
        scratch += [
            pltpu.VMEM((1, CH), jnp.int32),
            pltpu.VMEM((GI, 128), jnp.int32),
            pltpu.VMEM((CH, D), jnp.float32),
            pltpu.VMEM((CH * K, D), jnp.float32),
            pltpu.VMEM((CH, D), jnp.float32),
            pltpu.VMEM((CH, D), jnp.float32),
            pltpu.VMEM((CH, D), jnp.float32),
        ]
    scratch += [
        pltpu.VMEM((CH, D), jnp.float32),
        pltpu.VMEM((1, L), jnp.float32),
        pltpu.VMEM((1, L), jnp.float32),
        pltpu.SemaphoreType.DMA,
        pltpu.SemaphoreType.DMA,
    ]

    @functools.partial(
        pl.kernel,
        out_type=(jax.ShapeDtypeStruct((NW, L), jnp.float32),
                  jax.ShapeDtypeStruct((NW, L), jnp.float32)),
        mesh=mesh,
        scratch_types=scratch,
        compiler_params=pltpu.CompilerParams(
            use_tc_tiling_on_sc=False, needs_layout_passes=False),
    )
    def body(x_ref, xn_ref, xo_ref, w_ref, m_ref, e_ref, out1_ref, out2_ref,
             xi0, xni0, ex0, nb0, xob0, wpb0, mpb0,
             xi1, xni1, ex1, nb1, xob1, wpb1, mpb1,
             coef, res1, res2, sem0, sem1):
        slots = ((xi0, xni0, ex0, nb0, xob0, wpb0, mpb0, sem0),
                 (xi1, xni1, ex1, nb1, xob1, wpb1, mpb1, sem1))
        wid = lax.axis_index("s") * NC + lax.axis_index("c")

        def issue(g, slot):
            xi, xni, ex, nb, xob, wpb, mpb, sem = slots[slot]
            row0 = wid * BPW + g * CH
            pltpu.sync_copy(x_ref.at[pl.ds(wid * NG + g, 1)], xi)
            pltpu.sync_copy(xn_ref.at[pl.ds(wid * (NG * GI) + g * GI, GI)],
                            xni)
            for j in range(GI):
                for h in range(8):
                    v = xni[j, pl.ds(h * L, L)]
                    xni[j, pl.ds(h * L, L)] = (
                        (v & -32768) + ((v & 8191) << 2) + ((v >> 13) & 3))
            for h in range(CH // L):
                v = xi[0, pl.ds(h * L, L)]
                xi[0, pl.ds(h * L, L)] = (
                    (v & -32768) + ((v & 8191) << 2) + ((v >> 13) & 3))
            pltpu.sync_copy(xo_ref.at[pl.ds(row0, CH)], xob)
            pltpu.sync_copy(w_ref.at[pl.ds(row0, CH)], wpb)
            pltpu.sync_copy(m_ref.at[pl.ds(row0, CH)], mpb)
            descs = [
                pltpu.async_copy(e_ref.at[xni.at[j]],
                                 nb.at[pl.ds(j * 128, 128)], sem)
                for j in range(GI)
            ]
            descs.append(pltpu.async_copy(e_ref.at[xi.at[0]], ex, sem))
            return descs

        def compute(slot, accs):
            _, _, ex, nb, xob, wpb, mpb, _ = slots[slot]

            def cbody(r, carry):
                for h in range(2):
                    wv = wpb[r, pl.ds(h * L, L)]
                    mv = mpb[r, pl.ds(h * L, L)]
                    coef[r, pl.ds(h * L, L)] = wv * mv * mv
                return carry
            lax.fori_loop(0, CH, cbody, 0, unroll=4)

            def bbody(b, accs):
                a1_0, a1_1 = accs[0], accs[1]
                a2 = list(accs[2:])
                e0 = ex[b, pl.ds(0, L)]
                e1 = ex[b, pl.ds(L, L)]
                o0 = xob[b, pl.ds(0, L)]
                o1 = xob[b, pl.ds(L, L)]
                d0 = e0 - o0
                d1 = e1 - o1
                a1_0 = a1_0 + d0 * d0
                a1_1 = a1_1 + d1 * d1
                bb = jnp.broadcast_to(b, (L,))
                for k in range(K):
                    kk = jnp.full((L,), k, jnp.int32)
                    cv = plsc.load_gather(coef, [bb, kk])
                    r = b * K + k
                    n0 = nb[r, pl.ds(0, L)]
                    n1 = nb[r, pl.ds(L, L)]
                    t0 = e0 - n0
                    t1 = e1 - n1
                    a2[k % 4] = a2[k % 4] + t0 * t0 * cv
                    a2[4 + k % 4] = a2[4 + k % 4] + t1 * t1 * cv
                return (a1_0, a1_1) + tuple(a2)

            return lax.fori_loop(0, CH, bbody, accs)

        z = jnp.zeros((L,), jnp.float32)
        accs = (z,) * 10
        descs = issue(0, 0)
        for g in range(NG):
            slot = g & 1
            nxt = issue(g + 1, slot ^ 1) if g + 1 < NG else None
            for dsc in descs:
                dsc.wait()
            accs = compute(slot, accs)
            descs = nxt

        a1 = accs[0] + accs[1]
        a2 = accs[2]
        for v in accs[3:]:
            a2 = a2 + v
        res1[0, :] = a1
        res2[0, :] = a2
        pltpu.sync_copy(res1, out1_ref.at[pl.ds(wid, 1)])
        pltpu.sync_copy(res2, out2_ref.at[pl.ds(wid, 1)])

    return body(x2d, xn2d, x_original, wpad, mpad, E)


def _transpose_body(et_ref, out_ref):
    r = lax.broadcasted_iota(jnp.int32, (32, 32), 0)
    c = lax.broadcasted_iota(jnp.int32, (32, 32), 1)
    ident = (r == c).astype(jnp.float32)
    for u in range(4):
        out_ref[:, pl.ds(u * 32, 32)] = lax.dot_general(
            et_ref[:, pl.ds(u * 8192, 8192)], ident,
            (((0,), (0,)), ((), ())),
            preferred_element_type=jnp.float32)


def _transpose(ET):
    return pl.pallas_call(
        _transpose_body,
        grid=(31,),
        in_specs=[pl.BlockSpec((32, 32768), lambda j: (0, j))],
        out_specs=pl.BlockSpec((8192, 128), lambda j: (j, 0)),
        out_shape=jax.ShapeDtypeStruct((253952, 128), jnp.float32),
        compiler_params=pltpu.CompilerParams(
            dimension_semantics=("parallel",)),
    )(ET)


def _finish_body(alpha_ref, p1_ref, p2_ref, out_ref):
    s1 = jnp.sum(p1_ref[...])
    s2 = jnp.sum(p2_ref[...])
    out_ref[0, 0] = (alpha_ref[0] * s1 + s2) * (1.0 / B)


def _finish(alpha, p1, p2):
    return pl.pallas_call(
        _finish_body,
        out_shape=jax.ShapeDtypeStruct((1, 1), jnp.float32),
        in_specs=[
            pl.BlockSpec(memory_space=pltpu.SMEM),
            pl.BlockSpec(memory_space=pltpu.VMEM),
            pl.BlockSpec(memory_space=pltpu.VMEM),
        ],
        out_specs=pl.BlockSpec(memory_space=pltpu.SMEM),
    )(alpha, p1, p2)


def kernel(x, x_original, x_neighbor, weight, alpha, num_neighbor, mask, E):
    del num_neighbor
    x2d = x.reshape(B // CH, CH).astype(jnp.int32)
    xn2d = x_neighbor.reshape(B * K // 128, 128).astype(jnp.int32)
    wpad = jnp.pad(weight, ((0, 0), (0, D - K)))
    mpad = jnp.pad(mask, ((0, 0), (0, D - K)))
    E2 = _transpose(E.T).reshape(1015808, 32)
    p1, p2 = _sc_partials(x2d, xn2d, x_original, wpad, mpad, E2)
    out = _finish(alpha.reshape(1), p1, p2)
    return out[0, 0]
